# baseline (speedup 1.0000x reference)
import functools

import jax
import jax.numpy as jnp
from jax.experimental import pallas as pl
from jax.experimental.pallas import tpu as pltpu

_MASK_TOKEN_ID = 3
_MASK_RATIO = 0.3
_REGION_K = 14
_B = 1024
_N = 49
_H = 1024
_L = 20
_BB = 32


def _fused_kernel(q_ref, s_ref, attn_ref, img_ref, qout_ref, imgout_ref):
    a = attn_ref[...]
    ai = a[:, :, None]
    aj = a[:, None, :]
    idx = jax.lax.broadcasted_iota(jnp.int32, (1, _N, _N), 1)
    jdx = jax.lax.broadcasted_iota(jnp.int32, (1, _N, _N), 2)
    outranks = (aj > ai) | ((aj == ai) & (jdx < idx))
    vrank = outranks.astype(jnp.int32).sum(axis=2)
    vmask = vrank < _REGION_K
    imgout_ref[...] = jnp.where(vmask[:, :, None], 0.0, img_ref[...])

    q = q_ref[...]
    s = s_ref[...]
    content = q > 3
    si = s[:, :, None]
    sj = s[:, None, :]
    li = jax.lax.broadcasted_iota(jnp.int32, (1, _L, _L), 1)
    lj = jax.lax.broadcasted_iota(jnp.int32, (1, _L, _L), 2)
    precedes = (sj < si) | ((sj == si) & (lj < li))
    precedes = precedes & content[:, None, :]
    lrank = precedes.astype(jnp.int32).sum(axis=2)
    n_content = content.astype(jnp.int32).sum(axis=1, keepdims=True)
    n_mask = jnp.maximum(
        1, jnp.floor(n_content.astype(jnp.float32) * _MASK_RATIO).astype(jnp.int32)
    )
    do_mask = content & (lrank < n_mask) & (n_content > 0)
    qout_ref[...] = jnp.where(do_mask, jnp.asarray(_MASK_TOKEN_ID, q.dtype), q)


@functools.partial(jax.jit, static_argnums=())
def _run(questions, img_features, attn_weights, scores):
    grid = (_B // _BB,)
    out = pl.pallas_call(
        _fused_kernel,
        grid=grid,
        in_specs=[
            pl.BlockSpec((_BB, _L), lambda b: (b, 0)),
            pl.BlockSpec((_BB, _L), lambda b: (b, 0)),
            pl.BlockSpec((_BB, _N), lambda b: (b, 0)),
            pl.BlockSpec((_BB, _N, _H), lambda b: (b, 0, 0)),
        ],
        out_specs=[
            pl.BlockSpec((_BB, _L), lambda b: (b, 0)),
            pl.BlockSpec((_BB, _N, _H), lambda b: (b, 0, 0)),
        ],
        out_shape=[
            jax.ShapeDtypeStruct((_B, _L), questions.dtype),
            jax.ShapeDtypeStruct((_B, _N, _H), img_features.dtype),
        ],
        compiler_params=pltpu.CompilerParams(
            dimension_semantics=("arbitrary",),
        ),
    )(questions, scores, attn_weights, img_features)
    return out[1], out[0]


def kernel(questions, img_features, attn_weights):
    key = jax.random.fold_in(jax.random.key(0), 12345)
    scores = jax.random.uniform(key, (_B, _L))
    cf_img, cf_q = _run(questions, img_features, attn_weights, scores)
    return (cf_img, cf_q)


# speedup vs baseline: 3.6931x; 3.6931x over previous
import jax
import jax.numpy as jnp
from jax.experimental import pallas as pl
from jax.experimental.pallas import tpu as pltpu

_MASK_TOKEN_ID = 3
_MASK_RATIO = 0.3
_REGION_K = 14
_B = 1024
_N = 49
_H = 1024
_L = 20
_BB = 32


def _fused_kernel(q_ref, s_ref, attn_ref, img_ref, qout_ref, imgout_ref):
    a = attn_ref[...]
    ai = a[:, :, None]
    aj = a[:, None, :]
    idx = jax.lax.broadcasted_iota(jnp.int32, (1, _N, _N), 1)
    jdx = jax.lax.broadcasted_iota(jnp.int32, (1, _N, _N), 2)
    outranks = (aj > ai) | ((aj == ai) & (jdx < idx))
    vrank = outranks.astype(jnp.int32).sum(axis=2)
    keep = (vrank >= _REGION_K).astype(jnp.float32)
    row = jax.lax.broadcasted_iota(jnp.int32, (_BB, _BB), 0)
    col = jax.lax.broadcasted_iota(jnp.int32, (_BB, _BB), 1)
    eye = (row == col).astype(jnp.float32)
    keep_t = jax.lax.dot_general(
        keep, eye, ((( 0,), (0,)), ((), ())),
        preferred_element_type=jnp.float32,
    )
    imgout_ref[...] = img_ref[...] * keep_t[:, :, None]

    q = q_ref[...]
    s = s_ref[...]
    content = q > 3
    si = s[:, :, None]
    sj = s[:, None, :]
    li = jax.lax.broadcasted_iota(jnp.int32, (1, _L, _L), 1)
    lj = jax.lax.broadcasted_iota(jnp.int32, (1, _L, _L), 2)
    precedes = (sj < si) | ((sj == si) & (lj < li))
    precedes = precedes & content[:, None, :]
    lrank = precedes.astype(jnp.int32).sum(axis=2)
    n_content = content.astype(jnp.int32).sum(axis=1, keepdims=True)
    n_mask = jnp.maximum(
        1, jnp.floor(n_content.astype(jnp.float32) * _MASK_RATIO).astype(jnp.int32)
    )
    do_mask = content & (lrank < n_mask) & (n_content > 0)
    qout_ref[...] = jnp.where(do_mask, jnp.asarray(_MASK_TOKEN_ID, q.dtype), q)


def _run(questions, scores, attn, img_t):
    grid = (_B // _BB,)
    qout, imgout = pl.pallas_call(
        _fused_kernel,
        grid=grid,
        in_specs=[
            pl.BlockSpec((_BB, _L), lambda b: (b, 0)),
            pl.BlockSpec((_BB, _L), lambda b: (b, 0)),
            pl.BlockSpec((_BB, _N), lambda b: (b, 0)),
            pl.BlockSpec((_N, _BB, _H), lambda b: (0, b, 0)),
        ],
        out_specs=[
            pl.BlockSpec((_BB, _L), lambda b: (b, 0)),
            pl.BlockSpec((_N, _BB, _H), lambda b: (0, b, 0)),
        ],
        out_shape=[
            jax.ShapeDtypeStruct((_B, _L), questions.dtype),
            jax.ShapeDtypeStruct((_N, _B, _H), img_t.dtype),
        ],
        compiler_params=pltpu.CompilerParams(
            dimension_semantics=("arbitrary",),
        ),
    )(questions, scores, attn, img_t)
    return qout, imgout


def kernel(questions, img_features, attn_weights):
    key = jax.random.fold_in(jax.random.key(0), 12345)
    scores = jax.random.uniform(key, (_B, _L))
    qout, imgout_t = _run(
        questions,
        scores,
        attn_weights,
        jnp.transpose(img_features, (1, 0, 2)),
    )
    return (jnp.transpose(imgout_t, (1, 0, 2)), qout)


# speedup vs baseline: 3.9017x; 1.0565x over previous
import jax
import jax.numpy as jnp
from jax.experimental import pallas as pl
from jax.experimental.pallas import tpu as pltpu

_MASK_TOKEN_ID = 3
_MASK_RATIO = 0.3
_REGION_K = 14
_B = 1024
_N = 49
_H = 1024
_L = 20
_BB = 64


def _fused_kernel(q_ref, s_ref, attn_ref, img_ref, qout_ref, imgout_ref):
    a = attn_ref[...]
    ai = a[:, :, None]
    aj = a[:, None, :]
    idx = jax.lax.broadcasted_iota(jnp.int32, (1, _N, _N), 1)
    jdx = jax.lax.broadcasted_iota(jnp.int32, (1, _N, _N), 2)
    outranks = (aj > ai) | ((aj == ai) & (jdx < idx))
    vrank = outranks.astype(jnp.int32).sum(axis=2)
    keep = (vrank >= _REGION_K).astype(jnp.float32)
    row = jax.lax.broadcasted_iota(jnp.int32, (_BB, _BB), 0)
    col = jax.lax.broadcasted_iota(jnp.int32, (_BB, _BB), 1)
    eye = (row == col).astype(jnp.float32)
    keep_t = jax.lax.dot_general(
        keep, eye, ((( 0,), (0,)), ((), ())),
        preferred_element_type=jnp.float32,
    )
    imgout_ref[...] = img_ref[...] * keep_t[:, :, None]

    q = q_ref[...]
    s = s_ref[...]
    content = q > 3
    si = s[:, :, None]
    sj = s[:, None, :]
    li = jax.lax.broadcasted_iota(jnp.int32, (1, _L, _L), 1)
    lj = jax.lax.broadcasted_iota(jnp.int32, (1, _L, _L), 2)
    precedes = (sj < si) | ((sj == si) & (lj < li))
    precedes = precedes & content[:, None, :]
    lrank = precedes.astype(jnp.int32).sum(axis=2)
    n_content = content.astype(jnp.int32).sum(axis=1, keepdims=True)
    n_mask = jnp.maximum(
        1, jnp.floor(n_content.astype(jnp.float32) * _MASK_RATIO).astype(jnp.int32)
    )
    do_mask = content & (lrank < n_mask) & (n_content > 0)
    qout_ref[...] = jnp.where(do_mask, jnp.asarray(_MASK_TOKEN_ID, q.dtype), q)


def _run(questions, scores, attn, img_t):
    grid = (_B // _BB,)
    qout, imgout = pl.pallas_call(
        _fused_kernel,
        grid=grid,
        in_specs=[
            pl.BlockSpec((_BB, _L), lambda b: (b, 0)),
            pl.BlockSpec((_BB, _L), lambda b: (b, 0)),
            pl.BlockSpec((_BB, _N), lambda b: (b, 0)),
            pl.BlockSpec((_N, _BB, _H), lambda b: (0, b, 0)),
        ],
        out_specs=[
            pl.BlockSpec((_BB, _L), lambda b: (b, 0)),
            pl.BlockSpec((_N, _BB, _H), lambda b: (0, b, 0)),
        ],
        out_shape=[
            jax.ShapeDtypeStruct((_B, _L), questions.dtype),
            jax.ShapeDtypeStruct((_N, _B, _H), img_t.dtype),
        ],
        compiler_params=pltpu.CompilerParams(
            dimension_semantics=("arbitrary",),
        ),
    )(questions, scores, attn, img_t)
    return qout, imgout


def kernel(questions, img_features, attn_weights):
    key = jax.random.fold_in(jax.random.key(0), 12345)
    scores = jax.random.uniform(key, (_B, _L))
    qout, imgout_t = _run(
        questions,
        scores,
        attn_weights,
        jnp.transpose(img_features, (1, 0, 2)),
    )
    return (jnp.transpose(imgout_t, (1, 0, 2)), qout)
